# Initial kernel scaffold; baseline (speedup 1.0000x reference)
import jax
import jax.numpy as jnp
from jax.experimental import pallas as pl


def kernel(x, edge_index, edge_weight, W1, b1, W2, b2):
    raise NotImplementedError("write your pallas kernel here")


# speedup vs baseline: 6.1716x; 6.1716x over previous
import functools

import jax
import jax.numpy as jnp
from jax import lax
from jax.experimental import pallas as pl
from jax.experimental.pallas import tpu as pltpu
from jax.experimental.pallas import tpu_sc as plsc

NC = 2
NS = 16
NW = NC * NS
LANES = 16
K_BLK = 80


def _tile_slices(n):
    ch = (((n + NS - 1) // NS) + 7) // 8 * 8
    last = n - (NS - 1) * ch
    assert 0 < last <= ch and ch % 8 == 0 and last % 8 == 0
    return ch, last


def _lane_bcast(vec, lane):
    idx = jnp.full((LANES, 1), lane, jnp.int32)
    dnums = lax.GatherDimensionNumbers(
        offset_dims=(), collapsed_slice_dims=(0,), start_index_map=(0,))
    return lax.gather(vec, idx, dnums, (1,),
                      mode=lax.GatherScatterMode.PROMISE_IN_BOUNDS)


def _zero_vmem_2d(ref, rows, d):
    zero16 = jnp.zeros((LANES,), jnp.float32)

    def body(r, carry):
        for q in range(d // LANES):
            ref[r, pl.ds(q * LANES, LANES)] = zero16
        return carry

    lax.fori_loop(0, rows, body, 0)


def _zero_vmem_1d(ref, total):
    zero16 = jnp.zeros((LANES,), jnp.float32)

    def body(i, carry):
        ref[pl.ds(i * LANES, LANES)] = zero16
        return carry

    lax.fori_loop(0, total // LANES, body, 0)


def _make_deg_kernel(n, e):
    per_w = e // NW
    nblk = per_w // K_BLK
    assert per_w * NW == e and nblk * K_BLK == per_w
    ch, last = _tile_slices(n)
    mesh = plsc.VectorSubcoreMesh(core_axis_name="c", subcore_axis_name="s")

    @functools.partial(
        pl.kernel,
        out_type=jax.ShapeDtypeStruct((NC * n,), jnp.float32),
        mesh=mesh,
        scratch_types=[
            pltpu.VMEM((K_BLK,), jnp.int32),
            pltpu.VMEM((K_BLK,), jnp.float32),
            pltpu.VMEM(((ch + LANES - 1) // LANES * LANES,), jnp.float32),
            pltpu.VMEM_SHARED((n,), jnp.float32),
        ],
    )
    def deg_kernel(col_hbm, w_hbm, out_hbm, col_v, w_v, zed_v, acc_sh):
        c = lax.axis_index("c")
        s = lax.axis_index("s")
        wid = c * NS + s

        _zero_vmem_1d(zed_v, (ch + LANES - 1) // LANES * LANES)

        @pl.when(s < NS - 1)
        def _():
            pltpu.sync_copy(zed_v.at[pl.ds(0, ch)], acc_sh.at[pl.ds(s * ch, ch)])

        @pl.when(s == NS - 1)
        def _():
            pltpu.sync_copy(zed_v.at[pl.ds(0, last)],
                            acc_sh.at[pl.ds((NS - 1) * ch, last)])

        plsc.subcore_barrier()
        base = wid * per_w

        def blk(j, carry):
            off = base + j * K_BLK
            pltpu.sync_copy(col_hbm.at[pl.ds(off, K_BLK)], col_v)
            pltpu.sync_copy(w_hbm.at[pl.ds(off, K_BLK)], w_v)
            pltpu.sync_copy(w_v, acc_sh.at[col_v], add=True)
            return carry

        lax.fori_loop(0, nblk, blk, 0)
        plsc.subcore_barrier()

        @pl.when(s < NS - 1)
        def _():
            pltpu.sync_copy(acc_sh.at[pl.ds(s * ch, ch)], zed_v.at[pl.ds(0, ch)])
            pltpu.sync_copy(zed_v.at[pl.ds(0, ch)],
                            out_hbm.at[pl.ds(c * n + s * ch, ch)])

        @pl.when(s == NS - 1)
        def _():
            pltpu.sync_copy(acc_sh.at[pl.ds((NS - 1) * ch, last)],
                            zed_v.at[pl.ds(0, last)])
            pltpu.sync_copy(zed_v.at[pl.ds(0, last)],
                            out_hbm.at[pl.ds(c * n + (NS - 1) * ch, last)])

    return deg_kernel


def _make_edge_kernel(n, e, d):
    per_w = e // NW
    nblk = per_w // K_BLK
    assert per_w * NW == e and nblk * K_BLK == per_w and d % LANES == 0
    ch, last = _tile_slices(n)
    mesh = plsc.VectorSubcoreMesh(core_axis_name="c", subcore_axis_name="s")

    @functools.partial(
        pl.kernel,
        out_type=jax.ShapeDtypeStruct((NC * n, d), jnp.float32),
        mesh=mesh,
        scratch_types=[
            pltpu.VMEM((K_BLK,), jnp.int32),
            pltpu.VMEM((K_BLK,), jnp.int32),
            pltpu.VMEM((K_BLK,), jnp.float32),
            pltpu.VMEM((K_BLK, d), jnp.float32),
            pltpu.VMEM((ch, d), jnp.float32),
            pltpu.VMEM_SHARED((n, d), jnp.float32),
            pltpu.SemaphoreType.DMA,
        ],
        compiler_params=pltpu.CompilerParams(use_tc_tiling_on_sc=False),
    )
    def edge_kernel(g_hbm, row_hbm, col_hbm, w_hbm, out_hbm,
                    row_v, col_v, w_v, rows_v, zed_v, acc_sh, sem):
        c = lax.axis_index("c")
        s = lax.axis_index("s")
        wid = c * NS + s

        _zero_vmem_2d(zed_v, ch, d)

        @pl.when(s < NS - 1)
        def _():
            pltpu.sync_copy(zed_v, acc_sh.at[pl.ds(s * ch, ch)])

        @pl.when(s == NS - 1)
        def _():
            pltpu.sync_copy(zed_v.at[pl.ds(0, last)],
                            acc_sh.at[pl.ds((NS - 1) * ch, last)])

        plsc.subcore_barrier()
        base = wid * per_w

        def blk(j, carry):
            off = base + j * K_BLK
            pltpu.sync_copy(row_hbm.at[pl.ds(off, K_BLK)], row_v)
            pltpu.sync_copy(col_hbm.at[pl.ds(off, K_BLK)], col_v)
            pltpu.sync_copy(w_hbm.at[pl.ds(off, K_BLK)], w_v)
            pltpu.async_copy(g_hbm.at[row_v], rows_v, sem).wait()

            def scale(kb, carry2):
                w16 = w_v[pl.ds(kb * LANES, LANES)]
                for jj in range(LANES):
                    wk = _lane_bcast(w16, jj)
                    k = kb * LANES + jj
                    for dd in range(d // LANES):
                        sl = pl.ds(dd * LANES, LANES)
                        rows_v[k, sl] = rows_v[k, sl] * wk
                return carry2

            lax.fori_loop(0, K_BLK // LANES, scale, 0)
            pltpu.sync_copy(rows_v, acc_sh.at[col_v], add=True)
            return carry

        lax.fori_loop(0, nblk, blk, 0)
        plsc.subcore_barrier()

        @pl.when(s < NS - 1)
        def _():
            pltpu.sync_copy(acc_sh.at[pl.ds(s * ch, ch)], zed_v)
            pltpu.sync_copy(zed_v, out_hbm.at[pl.ds(c * n + s * ch, ch)])

        @pl.when(s == NS - 1)
        def _():
            pltpu.sync_copy(acc_sh.at[pl.ds((NS - 1) * ch, last)],
                            zed_v.at[pl.ds(0, last)])
            pltpu.sync_copy(zed_v.at[pl.ds(0, last)],
                            out_hbm.at[pl.ds(c * n + (NS - 1) * ch, last)])

    return edge_kernel


def _dis_from(deg_ref):
    deg = deg_ref[:, 0] + deg_ref[:, 1] + 1.0
    return jnp.where(deg > 0, lax.rsqrt(deg), 0.0)


def _prep_body(deg_ref, x_ref, w_ref, g_ref):
    dis = _dis_from(deg_ref)
    h = jnp.dot(x_ref[...], w_ref[...], preferred_element_type=jnp.float32)
    g_ref[...] = h * dis[:, None]


def _make_mid_body(nparts):
    def body(*refs):
        deg_ref = refs[0]
        s_refs = refs[1:1 + nparts]
        g1_ref, b1_ref, w2_ref, g2_ref = refs[1 + nparts:]
        dis = _dis_from(deg_ref)
        s = jnp.concatenate([r[0] + r[1] for r in s_refs], axis=-1)
        t = (s + g1_ref[...]) * dis[:, None] + b1_ref[...]
        t = jnp.maximum(t, 0.0)
        h2 = jnp.dot(t, w2_ref[...], preferred_element_type=jnp.float32)
        g2_ref[...] = h2 * dis[:, None]
    return body


def _fin_body(deg_ref, s_ref, g2_ref, b2_ref, o_ref):
    dis = _dis_from(deg_ref)
    o_ref[...] = (s_ref[0] + s_ref[1] + g2_ref[...]) * dis[:, None] + b2_ref[...]


def _row_blocks(n):
    for blk in (2000, 1000, 500, 250, 125, n):
        if n % blk == 0 and blk % 8 == 0:
            return blk, n // blk
    return n, 1


def kernel(x, edge_index, edge_weight, W1, b1, W2, b2):
    n, f = x.shape
    d1 = W1.shape[1]
    d2 = W2.shape[1]
    e = edge_weight.shape[0]

    row = edge_index[0].astype(jnp.int32)
    col = edge_index[1].astype(jnp.int32)
    w = edge_weight.astype(jnp.float32)

    emult = NW * K_BLK
    e_pad = (e + emult - 1) // emult * emult
    if e_pad != e:
        extra = e_pad - e
        pad_idx = (jnp.arange(extra, dtype=jnp.int32) * 16) % n
        row = jnp.concatenate([row, pad_idx])
        col = jnp.concatenate([col, pad_idx])
        w = jnp.concatenate([w, jnp.zeros((extra,), jnp.float32)])

    deg_parts = _make_deg_kernel(n, e_pad)(col, w)
    deg_nt = deg_parts.reshape(NC, n).T

    blk, nblk = _row_blocks(n)
    full2 = lambda i: (0, 0)

    g1 = pl.pallas_call(
        _prep_body,
        grid=(nblk,),
        in_specs=[
            pl.BlockSpec((blk, NC), lambda i: (i, 0)),
            pl.BlockSpec((blk, f), lambda i: (i, 0)),
            pl.BlockSpec((f, d1), full2),
        ],
        out_specs=pl.BlockSpec((blk, d1), lambda i: (i, 0)),
        out_shape=jax.ShapeDtypeStruct((n, d1), jnp.float32),
    )(deg_nt, x, W1)

    DCH = 64
    edge64 = _make_edge_kernel(n, e_pad, DCH)
    s1_parts = [edge64(g1[:, i * DCH:(i + 1) * DCH], row, col, w)
                .reshape(NC, n, DCH) for i in range(d1 // DCH)]

    nparts = d1 // DCH
    g2 = pl.pallas_call(
        _make_mid_body(nparts),
        grid=(nblk,),
        in_specs=[
            pl.BlockSpec((blk, NC), lambda i: (i, 0)),
            *[pl.BlockSpec((NC, blk, DCH), lambda i: (0, i, 0))
              for _ in range(nparts)],
            pl.BlockSpec((blk, d1), lambda i: (i, 0)),
            pl.BlockSpec((1, d1), full2),
            pl.BlockSpec((d1, d2), full2),
        ],
        out_specs=pl.BlockSpec((blk, d2), lambda i: (i, 0)),
        out_shape=jax.ShapeDtypeStruct((n, d2), jnp.float32),
    )(deg_nt, *s1_parts, g1, b1.reshape(1, d1), W2)

    edge2 = edge64 if d2 == DCH else _make_edge_kernel(n, e_pad, d2)
    s2 = edge2(g2, row, col, w).reshape(NC, n, d2)

    out = pl.pallas_call(
        _fin_body,
        grid=(nblk,),
        in_specs=[
            pl.BlockSpec((blk, NC), lambda i: (i, 0)),
            pl.BlockSpec((NC, blk, d2), lambda i: (0, i, 0)),
            pl.BlockSpec((blk, d2), lambda i: (i, 0)),
            pl.BlockSpec((1, d2), full2),
        ],
        out_specs=pl.BlockSpec((blk, d2), lambda i: (i, 0)),
        out_shape=jax.ShapeDtypeStruct((n, d2), jnp.float32),
    )(deg_nt, s2, g2, b2.reshape(1, d2))

    return out


# speedup vs baseline: 29.8067x; 4.8297x over previous
import functools

import jax
import jax.numpy as jnp
from jax import lax
from jax.experimental import pallas as pl
from jax.experimental.pallas import tpu as pltpu
from jax.experimental.pallas import tpu_sc as plsc

NC = 2
NS = 16
NW = NC * NS
LANES = 16
K_BLK = 128


def _tile_slices(n):
    ch = (((n + NS - 1) // NS) + 7) // 8 * 8
    last = n - (NS - 1) * ch
    assert 0 < last <= ch and ch % 8 == 0 and last % 8 == 0
    return ch, last


def _lane_bcast(vec, lane):
    idx = jnp.full((LANES, 1), lane, jnp.int32)
    dnums = lax.GatherDimensionNumbers(
        offset_dims=(), collapsed_slice_dims=(0,), start_index_map=(0,))
    return lax.gather(vec, idx, dnums, (1,),
                      mode=lax.GatherScatterMode.PROMISE_IN_BOUNDS)


def _zero_vmem_2d(ref, rows, d):
    zero16 = jnp.zeros((LANES,), jnp.float32)

    def body(r, carry):
        for q in range(d // LANES):
            ref[r, pl.ds(q * LANES, LANES)] = zero16
        return carry

    lax.fori_loop(0, rows, body, 0)


def _zero_vmem_1d(ref, total):
    zero16 = jnp.zeros((LANES,), jnp.float32)

    def body(i, carry):
        ref[pl.ds(i * LANES, LANES)] = zero16
        return carry

    lax.fori_loop(0, total // LANES, body, 0)


def _make_deg_kernel(n, nblk):
    ch, last = _tile_slices(n)
    chz = (ch + LANES - 1) // LANES * LANES
    mesh = plsc.VectorSubcoreMesh(core_axis_name="c", subcore_axis_name="s")

    @functools.partial(
        pl.kernel,
        out_type=jax.ShapeDtypeStruct((NC * n,), jnp.float32),
        mesh=mesh,
        scratch_types=[
            pltpu.VMEM((nblk, K_BLK), jnp.int32),
            pltpu.VMEM((nblk, K_BLK), jnp.float32),
            pltpu.VMEM((chz,), jnp.float32),
            pltpu.VMEM_SHARED((n,), jnp.float32),
            pltpu.SemaphoreType.DMA,
        ],
        compiler_params=pltpu.CompilerParams(use_tc_tiling_on_sc=False),
    )
    def deg_kernel(col_hbm, w_hbm, out_hbm, col_v, w_v, zed_v, acc_sh, sem):
        c = lax.axis_index("c")
        s = lax.axis_index("s")
        wid = c * NS + s

        _zero_vmem_1d(zed_v, chz)

        @pl.when(s < NS - 1)
        def _():
            pltpu.sync_copy(zed_v.at[pl.ds(0, ch)], acc_sh.at[pl.ds(s * ch, ch)])

        @pl.when(s == NS - 1)
        def _():
            pltpu.sync_copy(zed_v.at[pl.ds(0, last)],
                            acc_sh.at[pl.ds((NS - 1) * ch, last)])

        pltpu.sync_copy(col_hbm.at[wid], col_v)
        pltpu.sync_copy(w_hbm.at[wid], w_v)
        plsc.subcore_barrier()

        GRP = 8

        def grp(gg, carry):
            for b in range(GRP):
                pltpu.async_copy(w_v.at[gg * GRP + b],
                                 acc_sh.at[col_v.at[gg * GRP + b]], sem,
                                 add=True)
            for b in range(GRP):
                pltpu.make_async_copy(w_v.at[gg * GRP + b],
                                      acc_sh.at[col_v.at[gg * GRP + b]],
                                      sem).wait()
            return carry

        assert nblk % GRP == 0
        lax.fori_loop(0, nblk // GRP, grp, 0)
        plsc.subcore_barrier()

        @pl.when(s < NS - 1)
        def _():
            pltpu.sync_copy(acc_sh.at[pl.ds(s * ch, ch)], zed_v.at[pl.ds(0, ch)])
            pltpu.sync_copy(zed_v.at[pl.ds(0, ch)],
                            out_hbm.at[pl.ds(c * n + s * ch, ch)])

        @pl.when(s == NS - 1)
        def _():
            pltpu.sync_copy(acc_sh.at[pl.ds((NS - 1) * ch, last)],
                            zed_v.at[pl.ds(0, last)])
            pltpu.sync_copy(zed_v.at[pl.ds(0, last)],
                            out_hbm.at[pl.ds(c * n + (NS - 1) * ch, last)])

    return deg_kernel


def _chunk_list(total, zr):
    k, rem = divmod(total, zr)
    return [(i * zr, zr) for i in range(k)] + ([(k * zr, rem)] if rem else [])


ZR = 128


def _make_edge_kernel(n, nblk, d):
    assert d % LANES == 0 and nblk % 2 == 0
    ch, last = _tile_slices(n)
    mesh = plsc.VectorSubcoreMesh(core_axis_name="c", subcore_axis_name="s")

    @functools.partial(
        pl.kernel,
        out_type=jax.ShapeDtypeStruct((NC * n, d), jnp.float32),
        mesh=mesh,
        scratch_types=[
            pltpu.VMEM((nblk, K_BLK), jnp.int32),
            pltpu.VMEM((nblk, K_BLK), jnp.int32),
            pltpu.VMEM((nblk, K_BLK), jnp.float32),
            pltpu.VMEM((2, K_BLK, d), jnp.float32),
            pltpu.VMEM((2, K_BLK, d), jnp.float32),
            pltpu.VMEM((ZR, d), jnp.float32),
            pltpu.VMEM_SHARED((n, d), jnp.float32),
            pltpu.SemaphoreType.DMA,
            pltpu.SemaphoreType.DMA,
            pltpu.SemaphoreType.DMA,
            pltpu.SemaphoreType.DMA,
        ],
        compiler_params=pltpu.CompilerParams(use_tc_tiling_on_sc=False),
    )
    def edge_kernel(g_hbm, row_hbm, col_hbm, w_hbm, out_hbm,
                    row_v, col_v, w_v, rows_v, sc_v, zed_v, acc_sh,
                    gsem0, gsem1, ssem0, ssem1):
        gsem = (gsem0, gsem1)
        ssem = (ssem0, ssem1)
        c = lax.axis_index("c")
        s = lax.axis_index("s")
        wid = c * NS + s

        _zero_vmem_2d(zed_v, ZR, d)

        @pl.when(s < NS - 1)
        def _():
            for off, sz in _chunk_list(ch, ZR):
                pltpu.sync_copy(zed_v.at[pl.ds(0, sz)],
                                acc_sh.at[pl.ds(s * ch + off, sz)])

        @pl.when(s == NS - 1)
        def _():
            for off, sz in _chunk_list(last, ZR):
                pltpu.sync_copy(zed_v.at[pl.ds(0, sz)],
                                acc_sh.at[pl.ds((NS - 1) * ch + off, sz)])

        pltpu.sync_copy(row_hbm.at[wid], row_v)
        pltpu.sync_copy(col_hbm.at[wid], col_v)
        pltpu.sync_copy(w_hbm.at[wid], w_v)
        plsc.subcore_barrier()

        for b in range(2):
            pltpu.async_copy(g_hbm.at[row_v.at[b]], rows_v.at[b], gsem[b])

        def blk2(j0, carry):
            for b in range(2):
                j = j0 * 2 + b
                pltpu.make_async_copy(g_hbm.at[row_v.at[j]], rows_v.at[b],
                                      gsem[b]).wait()

                @pl.when(j0 > 0)
                def _():
                    jp = j - 2
                    pltpu.make_async_copy(sc_v.at[b],
                                          acc_sh.at[col_v.at[jp]],
                                          ssem[b]).wait()

                def scale(kb, carry2):
                    w16 = w_v[j, pl.ds(kb * LANES, LANES)]
                    for jj in range(LANES):
                        wk = _lane_bcast(w16, jj)
                        k = kb * LANES + jj
                        for dd in range(d // LANES):
                            sl = pl.ds(dd * LANES, LANES)
                            sc_v[b, k, sl] = rows_v[b, k, sl] * wk
                    return carry2

                lax.fori_loop(0, K_BLK // LANES, scale, 0)
                pltpu.async_copy(sc_v.at[b], acc_sh.at[col_v.at[j]],
                                 ssem[b], add=True)

                @pl.when(j + 2 < nblk)
                def _():
                    pltpu.async_copy(g_hbm.at[row_v.at[j + 2]], rows_v.at[b],
                                     gsem[b])
            return carry

        lax.fori_loop(0, nblk // 2, blk2, 0)
        for b in range(2):
            pltpu.make_async_copy(sc_v.at[b],
                                  acc_sh.at[col_v.at[nblk - 2 + b]],
                                  ssem[b]).wait()
        plsc.subcore_barrier()

        @pl.when(s < NS - 1)
        def _():
            for off, sz in _chunk_list(ch, ZR):
                pltpu.sync_copy(acc_sh.at[pl.ds(s * ch + off, sz)],
                                zed_v.at[pl.ds(0, sz)])
                pltpu.sync_copy(zed_v.at[pl.ds(0, sz)],
                                out_hbm.at[pl.ds(c * n + s * ch + off, sz)])

        @pl.when(s == NS - 1)
        def _():
            for off, sz in _chunk_list(last, ZR):
                pltpu.sync_copy(acc_sh.at[pl.ds((NS - 1) * ch + off, sz)],
                                zed_v.at[pl.ds(0, sz)])
                pltpu.sync_copy(zed_v.at[pl.ds(0, sz)],
                                out_hbm.at[pl.ds(c * n + (NS - 1) * ch + off, sz)])

    return edge_kernel


def _dis_from(deg_ref):
    deg = deg_ref[:, 0] + deg_ref[:, 1] + 1.0
    return jnp.where(deg > 0, lax.rsqrt(deg), 0.0)


def _prep_body(deg_ref, x_ref, w_ref, g_ref):
    dis = _dis_from(deg_ref)
    h = jnp.dot(x_ref[...], w_ref[...], preferred_element_type=jnp.float32)
    g_ref[...] = h * dis[:, None]


def _make_mid_body(nparts):
    def body(*refs):
        deg_ref = refs[0]
        s_refs = refs[1:1 + nparts]
        g1_ref, b1_ref, w2_ref, g2_ref = refs[1 + nparts:]
        dis = _dis_from(deg_ref)
        s = jnp.concatenate([r[0] + r[1] for r in s_refs], axis=-1)
        t = (s + g1_ref[...]) * dis[:, None] + b1_ref[...]
        t = jnp.maximum(t, 0.0)
        h2 = jnp.dot(t, w2_ref[...], preferred_element_type=jnp.float32)
        g2_ref[...] = h2 * dis[:, None]
    return body


def _fin_body(deg_ref, s_ref, g2_ref, b2_ref, o_ref):
    dis = _dis_from(deg_ref)
    o_ref[...] = (s_ref[0] + s_ref[1] + g2_ref[...]) * dis[:, None] + b2_ref[...]


def _row_blocks(n):
    for blk in (2000, 1000, 500, 250, 125, n):
        if n % blk == 0 and blk % 8 == 0:
            return blk, n // blk
    return n, 1


def kernel(x, edge_index, edge_weight, W1, b1, W2, b2):
    n, f = x.shape
    d1 = W1.shape[1]
    d2 = W2.shape[1]
    e = edge_weight.shape[0]

    row = edge_index[0].astype(jnp.int32)
    col = edge_index[1].astype(jnp.int32)
    w = edge_weight.astype(jnp.float32)

    emult = NW * K_BLK * 2
    e_pad = (e + emult - 1) // emult * emult
    if e_pad != e:
        extra = e_pad - e
        pad_idx = (jnp.arange(extra, dtype=jnp.int32) * 16) % n
        row = jnp.concatenate([row, pad_idx])
        col = jnp.concatenate([col, pad_idx])
        w = jnp.concatenate([w, jnp.zeros((extra,), jnp.float32)])
    nblk = e_pad // (NW * K_BLK)
    row3 = row.reshape(NW, nblk, K_BLK)
    col3 = col.reshape(NW, nblk, K_BLK)
    w3 = w.reshape(NW, nblk, K_BLK)

    deg_parts = _make_deg_kernel(n, nblk)(col3, w3)
    deg_nt = deg_parts.reshape(NC, n).T

    blk, nrblk = _row_blocks(n)
    full2 = lambda i: (0, 0)

    g1 = pl.pallas_call(
        _prep_body,
        grid=(nrblk,),
        in_specs=[
            pl.BlockSpec((blk, NC), lambda i: (i, 0)),
            pl.BlockSpec((blk, f), lambda i: (i, 0)),
            pl.BlockSpec((f, d1), full2),
        ],
        out_specs=pl.BlockSpec((blk, d1), lambda i: (i, 0)),
        out_shape=jax.ShapeDtypeStruct((n, d1), jnp.float32),
    )(deg_nt, x, W1)

    DCH = 64
    edge64 = _make_edge_kernel(n, nblk, DCH)
    s1_parts = [edge64(g1[:, i * DCH:(i + 1) * DCH], row3, col3, w3)
                .reshape(NC, n, DCH) for i in range(d1 // DCH)]

    nparts = d1 // DCH
    g2 = pl.pallas_call(
        _make_mid_body(nparts),
        grid=(nrblk,),
        in_specs=[
            pl.BlockSpec((blk, NC), lambda i: (i, 0)),
            *[pl.BlockSpec((NC, blk, DCH), lambda i: (0, i, 0))
              for _ in range(nparts)],
            pl.BlockSpec((blk, d1), lambda i: (i, 0)),
            pl.BlockSpec((1, d1), full2),
            pl.BlockSpec((d1, d2), full2),
        ],
        out_specs=pl.BlockSpec((blk, d2), lambda i: (i, 0)),
        out_shape=jax.ShapeDtypeStruct((n, d2), jnp.float32),
    )(deg_nt, *s1_parts, g1, b1.reshape(1, d1), W2)

    edge2 = edge64 if d2 == DCH else _make_edge_kernel(n, nblk, d2)
    s2 = edge2(g2, row3, col3, w3).reshape(NC, n, d2)

    out = pl.pallas_call(
        _fin_body,
        grid=(nrblk,),
        in_specs=[
            pl.BlockSpec((blk, NC), lambda i: (i, 0)),
            pl.BlockSpec((NC, blk, d2), lambda i: (0, i, 0)),
            pl.BlockSpec((blk, d2), lambda i: (i, 0)),
            pl.BlockSpec((1, d2), full2),
        ],
        out_specs=pl.BlockSpec((blk, d2), lambda i: (i, 0)),
        out_shape=jax.ShapeDtypeStruct((n, d2), jnp.float32),
    )(deg_nt, s2, g2, b2.reshape(1, d2))

    return out


# speedup vs baseline: 30.7936x; 1.0331x over previous
import functools

import jax
import jax.numpy as jnp
from jax import lax
from jax.experimental import pallas as pl
from jax.experimental.pallas import tpu as pltpu
from jax.experimental.pallas import tpu_sc as plsc

NC = 2
NS = 16
NW = NC * NS
LANES = 16
K_BLK = 128


def _tile_slices(n):
    ch = (((n + NS - 1) // NS) + 7) // 8 * 8
    last = n - (NS - 1) * ch
    assert 0 < last <= ch and ch % 8 == 0 and last % 8 == 0
    return ch, last


def _lane_bcast(vec, lane):
    idx = jnp.full((LANES, 1), lane, jnp.int32)
    dnums = lax.GatherDimensionNumbers(
        offset_dims=(), collapsed_slice_dims=(0,), start_index_map=(0,))
    return lax.gather(vec, idx, dnums, (1,),
                      mode=lax.GatherScatterMode.PROMISE_IN_BOUNDS)


def _zero_vmem_2d(ref, rows, d):
    zero16 = jnp.zeros((LANES,), jnp.float32)

    def body(r, carry):
        for q in range(d // LANES):
            ref[r, pl.ds(q * LANES, LANES)] = zero16
        return carry

    lax.fori_loop(0, rows, body, 0)


def _zero_vmem_1d(ref, total):
    zero16 = jnp.zeros((LANES,), jnp.float32)

    def body(i, carry):
        ref[pl.ds(i * LANES, LANES)] = zero16
        return carry

    lax.fori_loop(0, total // LANES, body, 0)


def _make_deg_kernel(n, nblk):
    ch, last = _tile_slices(n)
    chz = (ch + LANES - 1) // LANES * LANES
    mesh = plsc.VectorSubcoreMesh(core_axis_name="c", subcore_axis_name="s")

    @functools.partial(
        pl.kernel,
        out_type=jax.ShapeDtypeStruct((NC * n,), jnp.float32),
        mesh=mesh,
        scratch_types=[
            pltpu.VMEM((nblk, K_BLK), jnp.int32),
            pltpu.VMEM((nblk, K_BLK), jnp.float32),
            pltpu.VMEM((chz,), jnp.float32),
            pltpu.VMEM_SHARED((n,), jnp.float32),
            pltpu.SemaphoreType.DMA,
        ],
        compiler_params=pltpu.CompilerParams(use_tc_tiling_on_sc=False),
    )
    def deg_kernel(col_hbm, w_hbm, out_hbm, col_v, w_v, zed_v, acc_sh, sem):
        c = lax.axis_index("c")
        s = lax.axis_index("s")
        wid = c * NS + s

        _zero_vmem_1d(zed_v, chz)

        @pl.when(s < NS - 1)
        def _():
            pltpu.sync_copy(zed_v.at[pl.ds(0, ch)], acc_sh.at[pl.ds(s * ch, ch)])

        @pl.when(s == NS - 1)
        def _():
            pltpu.sync_copy(zed_v.at[pl.ds(0, last)],
                            acc_sh.at[pl.ds((NS - 1) * ch, last)])

        pltpu.sync_copy(col_hbm.at[wid], col_v)
        pltpu.sync_copy(w_hbm.at[wid], w_v)
        plsc.subcore_barrier()

        GRP = 8

        def grp(gg, carry):
            for b in range(GRP):
                pltpu.async_copy(w_v.at[gg * GRP + b],
                                 acc_sh.at[col_v.at[gg * GRP + b]], sem,
                                 add=True)
            for b in range(GRP):
                pltpu.make_async_copy(w_v.at[gg * GRP + b],
                                      acc_sh.at[col_v.at[gg * GRP + b]],
                                      sem).wait()
            return carry

        assert nblk % GRP == 0
        lax.fori_loop(0, nblk // GRP, grp, 0)
        plsc.subcore_barrier()

        @pl.when(s < NS - 1)
        def _():
            pltpu.sync_copy(acc_sh.at[pl.ds(s * ch, ch)], zed_v.at[pl.ds(0, ch)])
            pltpu.sync_copy(zed_v.at[pl.ds(0, ch)],
                            out_hbm.at[pl.ds(c * n + s * ch, ch)])

        @pl.when(s == NS - 1)
        def _():
            pltpu.sync_copy(acc_sh.at[pl.ds((NS - 1) * ch, last)],
                            zed_v.at[pl.ds(0, last)])
            pltpu.sync_copy(zed_v.at[pl.ds(0, last)],
                            out_hbm.at[pl.ds(c * n + (NS - 1) * ch, last)])

    return deg_kernel


def _chunk_list(total, zr):
    k, rem = divmod(total, zr)
    return [(i * zr, zr) for i in range(k)] + ([(k * zr, rem)] if rem else [])


ZR = 128


def _make_edge_kernel(n, nblk, d):
    assert d % LANES == 0 and nblk % 2 == 0
    ch, last = _tile_slices(n)
    mesh = plsc.VectorSubcoreMesh(core_axis_name="c", subcore_axis_name="s")

    @functools.partial(
        pl.kernel,
        out_type=jax.ShapeDtypeStruct((NC * n, d), jnp.float32),
        mesh=mesh,
        scratch_types=[
            pltpu.VMEM((nblk, K_BLK), jnp.int32),
            pltpu.VMEM((nblk, K_BLK), jnp.int32),
            pltpu.VMEM((nblk, K_BLK), jnp.float32),
            pltpu.VMEM((2, K_BLK, d), jnp.float32),
            pltpu.VMEM((2, K_BLK, d), jnp.float32),
            pltpu.VMEM((ZR, d), jnp.float32),
            pltpu.VMEM_SHARED((n, d), jnp.float32),
            pltpu.SemaphoreType.DMA,
            pltpu.SemaphoreType.DMA,
            pltpu.SemaphoreType.DMA,
            pltpu.SemaphoreType.DMA,
        ],
        compiler_params=pltpu.CompilerParams(use_tc_tiling_on_sc=False),
    )
    def edge_kernel(g_hbm, row_hbm, col_hbm, w_hbm, out_hbm,
                    row_v, col_v, w_v, rows_v, sc_v, zed_v, acc_sh,
                    gsem0, gsem1, ssem0, ssem1):
        gsem = (gsem0, gsem1)
        ssem = (ssem0, ssem1)
        c = lax.axis_index("c")
        s = lax.axis_index("s")
        wid = c * NS + s

        _zero_vmem_2d(zed_v, ZR, d)

        @pl.when(s < NS - 1)
        def _():
            for off, sz in _chunk_list(ch, ZR):
                pltpu.sync_copy(zed_v.at[pl.ds(0, sz)],
                                acc_sh.at[pl.ds(s * ch + off, sz)])

        @pl.when(s == NS - 1)
        def _():
            for off, sz in _chunk_list(last, ZR):
                pltpu.sync_copy(zed_v.at[pl.ds(0, sz)],
                                acc_sh.at[pl.ds((NS - 1) * ch + off, sz)])

        pltpu.sync_copy(row_hbm.at[wid], row_v)
        pltpu.sync_copy(col_hbm.at[wid], col_v)
        pltpu.sync_copy(w_hbm.at[wid], w_v)
        plsc.subcore_barrier()

        for b in range(2):
            pltpu.async_copy(g_hbm.at[row_v.at[b]], rows_v.at[b], gsem[b])

        def blk2(j0, carry):
            for b in range(2):
                j = j0 * 2 + b
                pltpu.make_async_copy(g_hbm.at[row_v.at[j]], rows_v.at[b],
                                      gsem[b]).wait()

                @pl.when(j0 > 0)
                def _():
                    jp = j - 2
                    pltpu.make_async_copy(sc_v.at[b],
                                          acc_sh.at[col_v.at[jp]],
                                          ssem[b]).wait()

                def scale(kb, carry2):
                    w16 = w_v[j, pl.ds(kb * LANES, LANES)]
                    for jj in range(LANES):
                        wk = _lane_bcast(w16, jj)
                        k = kb * LANES + jj
                        for dd in range(d // LANES):
                            sl = pl.ds(dd * LANES, LANES)
                            sc_v[b, k, sl] = rows_v[b, k, sl] * wk
                    return carry2

                lax.fori_loop(0, K_BLK // LANES, scale, 0)
                pltpu.async_copy(sc_v.at[b], acc_sh.at[col_v.at[j]],
                                 ssem[b], add=True)

                @pl.when(j + 2 < nblk)
                def _():
                    pltpu.async_copy(g_hbm.at[row_v.at[j + 2]], rows_v.at[b],
                                     gsem[b])
            return carry

        lax.fori_loop(0, nblk // 2, blk2, 0)
        for b in range(2):
            pltpu.make_async_copy(sc_v.at[b],
                                  acc_sh.at[col_v.at[nblk - 2 + b]],
                                  ssem[b]).wait()
        plsc.subcore_barrier()

        @pl.when(s < NS - 1)
        def _():
            for off, sz in _chunk_list(ch, ZR):
                pltpu.sync_copy(acc_sh.at[pl.ds(s * ch + off, sz)],
                                zed_v.at[pl.ds(0, sz)])
                pltpu.sync_copy(zed_v.at[pl.ds(0, sz)],
                                out_hbm.at[pl.ds(c * n + s * ch + off, sz)])

        @pl.when(s == NS - 1)
        def _():
            for off, sz in _chunk_list(last, ZR):
                pltpu.sync_copy(acc_sh.at[pl.ds((NS - 1) * ch + off, sz)],
                                zed_v.at[pl.ds(0, sz)])
                pltpu.sync_copy(zed_v.at[pl.ds(0, sz)],
                                out_hbm.at[pl.ds(c * n + (NS - 1) * ch + off, sz)])

    return edge_kernel


GI = 8


def _make_edge1_kernel(n, nblk2, d):
    assert d % LANES == 0 and nblk2 % (2 * GI) == 0 and nblk2 // GI >= 2
    ch, last = _tile_slices(n)
    ngrp = nblk2 // GI
    mesh = plsc.VectorSubcoreMesh(core_axis_name="c", subcore_axis_name="s")

    @functools.partial(
        pl.kernel,
        out_type=jax.ShapeDtypeStruct((NC * n, d), jnp.float32),
        mesh=mesh,
        scratch_types=[
            pltpu.VMEM((2, GI, K_BLK), jnp.int32),
            pltpu.VMEM((2, GI, K_BLK), jnp.int32),
            pltpu.VMEM((2, GI, K_BLK), jnp.float32),
            pltpu.VMEM((2, K_BLK, d), jnp.float32),
            pltpu.VMEM((2, K_BLK, d), jnp.float32),
            pltpu.VMEM((ZR, d), jnp.float32),
            pltpu.VMEM_SHARED((n, d), jnp.float32),
            pltpu.SemaphoreType.DMA,
            pltpu.SemaphoreType.DMA,
            pltpu.SemaphoreType.DMA,
            pltpu.SemaphoreType.DMA,
            pltpu.SemaphoreType.DMA,
            pltpu.SemaphoreType.DMA,
        ],
        compiler_params=pltpu.CompilerParams(use_tc_tiling_on_sc=False),
    )
    def edge1_kernel(g_hbm, row_hbm, col_hbm, w_hbm, out_hbm,
                     row_v, col_v, w_v, rows_v, sc_v, zed_v, acc_sh,
                     gsem0, gsem1, ssem0, ssem1, isem0, isem1):
        gsem = (gsem0, gsem1)
        ssem = (ssem0, ssem1)
        isem = (isem0, isem1)
        c = lax.axis_index("c")
        s = lax.axis_index("s")
        cn16 = jnp.full((LANES,), c * n, jnp.int32)

        _zero_vmem_2d(zed_v, ZR, d)

        @pl.when(s < NS - 1)
        def _():
            for off, sz in _chunk_list(ch, ZR):
                pltpu.sync_copy(zed_v.at[pl.ds(0, sz)],
                                acc_sh.at[pl.ds(s * ch + off, sz)])

        @pl.when(s == NS - 1)
        def _():
            for off, sz in _chunk_list(last, ZR):
                pltpu.sync_copy(zed_v.at[pl.ds(0, sz)],
                                acc_sh.at[pl.ds((NS - 1) * ch + off, sz)])

        def offset_rows(buf):
            def obody(q, carry):
                r = q // (K_BLK // LANES)
                o = (q % (K_BLK // LANES)) * LANES
                sl = pl.ds(o, LANES)
                row_v[buf, r, sl] = row_v[buf, r, sl] + cn16
                return carry
            lax.fori_loop(0, GI * K_BLK // LANES, obody, 0)

        def stage_group(g, buf, sem):
            pltpu.async_copy(row_hbm.at[s, pl.ds(g * GI, GI)], row_v.at[buf], sem)
            pltpu.async_copy(col_hbm.at[s, pl.ds(g * GI, GI)], col_v.at[buf], sem)
            pltpu.async_copy(w_hbm.at[s, pl.ds(g * GI, GI)], w_v.at[buf], sem)

        def wait_group(g, buf, sem):
            pltpu.make_async_copy(row_hbm.at[s, pl.ds(g * GI, GI)],
                                  row_v.at[buf], sem).wait()
            pltpu.make_async_copy(col_hbm.at[s, pl.ds(g * GI, GI)],
                                  col_v.at[buf], sem).wait()
            pltpu.make_async_copy(w_hbm.at[s, pl.ds(g * GI, GI)],
                                  w_v.at[buf], sem).wait()

        stage_group(0, 0, isem[0])
        wait_group(0, 0, isem[0])
        offset_rows(0)
        plsc.subcore_barrier()

        for b in range(2):
            pltpu.async_copy(g_hbm.at[row_v.at[0, b]], rows_v.at[b], gsem[b])

        def grouppair(g0, carry):
            for bgi in range(2):
                g = g0 * 2 + bgi
                for jb in range(GI):
                    b = jb % 2
                    j = g * GI + jb
                    pltpu.make_async_copy(g_hbm.at[row_v.at[bgi, jb]],
                                          rows_v.at[b], gsem[b]).wait()
                    if jb >= 2:
                        cprev = col_v.at[bgi, jb - 2]
                    else:
                        cprev = col_v.at[1 - bgi, GI - 2 + jb]

                    @pl.when(j >= 2)
                    def _(cprev=cprev, b=b):
                        pltpu.make_async_copy(sc_v.at[b], acc_sh.at[cprev],
                                              ssem[b]).wait()

                    def scale(kb, carry2, bgi=bgi, jb=jb, b=b):
                        w16 = w_v[bgi, jb, pl.ds(kb * LANES, LANES)]
                        for jj in range(LANES):
                            wk = _lane_bcast(w16, jj)
                            k = kb * LANES + jj
                            for dd in range(d // LANES):
                                sl = pl.ds(dd * LANES, LANES)
                                sc_v[b, k, sl] = rows_v[b, k, sl] * wk
                        return carry2

                    lax.fori_loop(0, K_BLK // LANES, scale, 0)
                    pltpu.async_copy(sc_v.at[b], acc_sh.at[col_v.at[bgi, jb]],
                                     ssem[b], add=True)

                    if jb == 2:
                        @pl.when(g + 1 < ngrp)
                        def _(g=g, bgi=bgi):
                            stage_group(g + 1, 1 - bgi, isem[1 - bgi])

                    if jb == GI - 2:
                        @pl.when(g + 1 < ngrp)
                        def _(g=g, bgi=bgi):
                            wait_group(g + 1, 1 - bgi, isem[1 - bgi])
                            offset_rows(1 - bgi)

                    if jb <= GI - 3:
                        nidx = row_v.at[bgi, jb + 2]
                    else:
                        nidx = row_v.at[1 - bgi, jb + 2 - GI]

                    @pl.when(j + 2 < nblk2)
                    def _(nidx=nidx, b=b):
                        pltpu.async_copy(g_hbm.at[nidx], rows_v.at[b], gsem[b])
            return carry

        lax.fori_loop(0, ngrp // 2, grouppair, 0)
        for b in range(2):
            pltpu.make_async_copy(sc_v.at[b],
                                  acc_sh.at[col_v.at[1, GI - 2 + b]],
                                  ssem[b]).wait()
        plsc.subcore_barrier()

        @pl.when(s < NS - 1)
        def _():
            for off, sz in _chunk_list(ch, ZR):
                pltpu.sync_copy(acc_sh.at[pl.ds(s * ch + off, sz)],
                                zed_v.at[pl.ds(0, sz)])
                pltpu.sync_copy(zed_v.at[pl.ds(0, sz)],
                                out_hbm.at[pl.ds(c * n + s * ch + off, sz)])

        @pl.when(s == NS - 1)
        def _():
            for off, sz in _chunk_list(last, ZR):
                pltpu.sync_copy(acc_sh.at[pl.ds((NS - 1) * ch + off, sz)],
                                zed_v.at[pl.ds(0, sz)])
                pltpu.sync_copy(zed_v.at[pl.ds(0, sz)],
                                out_hbm.at[pl.ds(c * n + (NS - 1) * ch + off, sz)])

    return edge1_kernel


def _dis_from(deg_ref):
    deg = deg_ref[:, 0] + deg_ref[:, 1] + 1.0
    return jnp.where(deg > 0, lax.rsqrt(deg), 0.0)


def _make_prep_body(nparts, dch):
    def body(deg_ref, x_ref, w_ref, g_ref):
        dis = _dis_from(deg_ref)
        h = jnp.dot(x_ref[...], w_ref[...], preferred_element_type=jnp.float32)
        g = h * dis[:, None]
        for p in range(nparts):
            g_ref[p] = g[:, p * dch:(p + 1) * dch]
    return body


def _make_mid_body(nparts):
    def body(deg_ref, s_ref, g1_ref, b1_ref, w2_ref, g2_ref):
        dis = _dis_from(deg_ref)
        t = jnp.concatenate([s_ref[p] + g1_ref[p] for p in range(nparts)],
                            axis=-1)
        t = t * dis[:, None] + b1_ref[...]
        t = jnp.maximum(t, 0.0)
        h2 = jnp.dot(t, w2_ref[...], preferred_element_type=jnp.float32)
        g2_ref[...] = h2 * dis[:, None]
    return body


def _fin_body(deg_ref, s_ref, g2_ref, b2_ref, o_ref):
    dis = _dis_from(deg_ref)
    o_ref[...] = (s_ref[0] + s_ref[1] + g2_ref[...]) * dis[:, None] + b2_ref[...]


def _row_blocks(n):
    for blk in (2000, 1000, 500, 250, 125, n):
        if n % blk == 0 and blk % 8 == 0:
            return blk, n // blk
    return n, 1


def kernel(x, edge_index, edge_weight, W1, b1, W2, b2):
    n, f = x.shape
    d1 = W1.shape[1]
    d2 = W2.shape[1]
    e = edge_weight.shape[0]

    row = edge_index[0].astype(jnp.int32)
    col = edge_index[1].astype(jnp.int32)
    w = edge_weight.astype(jnp.float32)

    emult = NS * K_BLK * GI * 2
    e_pad = (e + emult - 1) // emult * emult
    if e_pad != e:
        extra = e_pad - e
        pad_idx = (jnp.arange(extra, dtype=jnp.int32) * 16) % n
        row = jnp.concatenate([row, pad_idx])
        col = jnp.concatenate([col, pad_idx])
        w = jnp.concatenate([w, jnp.zeros((extra,), jnp.float32)])
    nblk = e_pad // (NW * K_BLK)
    nblk2 = e_pad // (NS * K_BLK)
    row3 = row.reshape(NW, nblk, K_BLK)
    col3 = col.reshape(NW, nblk, K_BLK)
    w3 = w.reshape(NW, nblk, K_BLK)
    row2 = row.reshape(NS, nblk2, K_BLK)
    col2 = col.reshape(NS, nblk2, K_BLK)
    w2 = w.reshape(NS, nblk2, K_BLK)

    deg_parts = _make_deg_kernel(n, nblk)(col3, w3)
    deg_nt = deg_parts.reshape(NC, n).T

    blk, nrblk = _row_blocks(n)
    full2 = lambda i: (0, 0)

    DCH = 64
    nparts = d1 // DCH
    assert nparts == NC and d1 == NC * DCH

    g1 = pl.pallas_call(
        _make_prep_body(nparts, DCH),
        grid=(nrblk,),
        in_specs=[
            pl.BlockSpec((blk, NC), lambda i: (i, 0)),
            pl.BlockSpec((blk, f), lambda i: (i, 0)),
            pl.BlockSpec((f, d1), full2),
        ],
        out_specs=pl.BlockSpec((nparts, blk, DCH), lambda i: (0, i, 0)),
        out_shape=jax.ShapeDtypeStruct((nparts, n, DCH), jnp.float32),
    )(deg_nt, x, W1)

    s1 = _make_edge1_kernel(n, nblk2, DCH)(
        g1.reshape(nparts * n, DCH), row2, col2, w2).reshape(NC, n, DCH)

    g2 = pl.pallas_call(
        _make_mid_body(nparts),
        grid=(nrblk,),
        in_specs=[
            pl.BlockSpec((blk, NC), lambda i: (i, 0)),
            pl.BlockSpec((NC, blk, DCH), lambda i: (0, i, 0)),
            pl.BlockSpec((nparts, blk, DCH), lambda i: (0, i, 0)),
            pl.BlockSpec((1, d1), full2),
            pl.BlockSpec((d1, d2), full2),
        ],
        out_specs=pl.BlockSpec((blk, d2), lambda i: (i, 0)),
        out_shape=jax.ShapeDtypeStruct((n, d2), jnp.float32),
    )(deg_nt, s1, g1, b1.reshape(1, d1), W2)

    s2 = _make_edge_kernel(n, nblk, d2)(g2, row3, col3, w3).reshape(NC, n, d2)

    out = pl.pallas_call(
        _fin_body,
        grid=(nrblk,),
        in_specs=[
            pl.BlockSpec((blk, NC), lambda i: (i, 0)),
            pl.BlockSpec((NC, blk, d2), lambda i: (0, i, 0)),
            pl.BlockSpec((blk, d2), lambda i: (i, 0)),
            pl.BlockSpec((1, d2), full2),
        ],
        out_specs=pl.BlockSpec((blk, d2), lambda i: (i, 0)),
        out_shape=jax.ShapeDtypeStruct((n, d2), jnp.float32),
    )(deg_nt, s2, g2, b2.reshape(1, d2))

    return out


# speedup vs baseline: 32.7245x; 1.0627x over previous
import functools

import jax
import jax.numpy as jnp
from jax import lax
from jax.experimental import pallas as pl
from jax.experimental.pallas import tpu as pltpu
from jax.experimental.pallas import tpu_sc as plsc

NC = 2
NS = 16
NW = NC * NS
LANES = 16
K_BLK = 128


def _tile_slices(n):
    ch = (((n + NS - 1) // NS) + 7) // 8 * 8
    last = n - (NS - 1) * ch
    assert 0 < last <= ch and ch % 8 == 0 and last % 8 == 0
    return ch, last


def _lane_bcast(vec, lane):
    idx = jnp.full((LANES, 1), lane, jnp.int32)
    dnums = lax.GatherDimensionNumbers(
        offset_dims=(), collapsed_slice_dims=(0,), start_index_map=(0,))
    return lax.gather(vec, idx, dnums, (1,),
                      mode=lax.GatherScatterMode.PROMISE_IN_BOUNDS)


def _zero_vmem_2d(ref, rows, d):
    zero16 = jnp.zeros((LANES,), jnp.float32)

    def body(r, carry):
        for q in range(d // LANES):
            ref[r, pl.ds(q * LANES, LANES)] = zero16
        return carry

    lax.fori_loop(0, rows, body, 0)


def _zero_vmem_1d(ref, total):
    zero16 = jnp.zeros((LANES,), jnp.float32)

    def body(i, carry):
        ref[pl.ds(i * LANES, LANES)] = zero16
        return carry

    lax.fori_loop(0, total // LANES, body, 0)


def _make_deg_kernel(n, nblk):
    ch, last = _tile_slices(n)
    chz = (ch + LANES - 1) // LANES * LANES
    mesh = plsc.VectorSubcoreMesh(core_axis_name="c", subcore_axis_name="s")

    @functools.partial(
        pl.kernel,
        out_type=jax.ShapeDtypeStruct((NC * n,), jnp.float32),
        mesh=mesh,
        scratch_types=[
            pltpu.VMEM((nblk, K_BLK), jnp.int32),
            pltpu.VMEM((nblk, K_BLK), jnp.float32),
            pltpu.VMEM((chz,), jnp.float32),
            pltpu.VMEM_SHARED((n,), jnp.float32),
            pltpu.SemaphoreType.DMA,
        ],
        compiler_params=pltpu.CompilerParams(use_tc_tiling_on_sc=False),
    )
    def deg_kernel(col_hbm, w_hbm, out_hbm, col_v, w_v, zed_v, acc_sh, sem):
        c = lax.axis_index("c")
        s = lax.axis_index("s")
        wid = c * NS + s

        _zero_vmem_1d(zed_v, chz)

        @pl.when(s < NS - 1)
        def _():
            pltpu.sync_copy(zed_v.at[pl.ds(0, ch)], acc_sh.at[pl.ds(s * ch, ch)])

        @pl.when(s == NS - 1)
        def _():
            pltpu.sync_copy(zed_v.at[pl.ds(0, last)],
                            acc_sh.at[pl.ds((NS - 1) * ch, last)])

        pltpu.sync_copy(col_hbm.at[wid], col_v)
        pltpu.sync_copy(w_hbm.at[wid], w_v)
        plsc.subcore_barrier()

        GRP = 8

        def grp(gg, carry):
            for b in range(GRP):
                pltpu.async_copy(w_v.at[gg * GRP + b],
                                 acc_sh.at[col_v.at[gg * GRP + b]], sem,
                                 add=True)
            for b in range(GRP):
                pltpu.make_async_copy(w_v.at[gg * GRP + b],
                                      acc_sh.at[col_v.at[gg * GRP + b]],
                                      sem).wait()
            return carry

        assert nblk % GRP == 0
        lax.fori_loop(0, nblk // GRP, grp, 0)
        plsc.subcore_barrier()

        @pl.when(s < NS - 1)
        def _():
            pltpu.sync_copy(acc_sh.at[pl.ds(s * ch, ch)], zed_v.at[pl.ds(0, ch)])
            pltpu.sync_copy(zed_v.at[pl.ds(0, ch)],
                            out_hbm.at[pl.ds(c * n + s * ch, ch)])

        @pl.when(s == NS - 1)
        def _():
            pltpu.sync_copy(acc_sh.at[pl.ds((NS - 1) * ch, last)],
                            zed_v.at[pl.ds(0, last)])
            pltpu.sync_copy(zed_v.at[pl.ds(0, last)],
                            out_hbm.at[pl.ds(c * n + (NS - 1) * ch, last)])

    return deg_kernel


def _chunk_list(total, zr):
    k, rem = divmod(total, zr)
    return [(i * zr, zr) for i in range(k)] + ([(k * zr, rem)] if rem else [])


ZR = 128


def _make_edge_kernel(n, nblk, d):
    assert d % LANES == 0 and nblk % 2 == 0
    ch, last = _tile_slices(n)
    mesh = plsc.VectorSubcoreMesh(core_axis_name="c", subcore_axis_name="s")

    @functools.partial(
        pl.kernel,
        out_type=jax.ShapeDtypeStruct((n, NC * d), jnp.float32),
        mesh=mesh,
        scratch_types=[
            pltpu.VMEM((nblk, K_BLK), jnp.int32),
            pltpu.VMEM((nblk, K_BLK), jnp.int32),
            pltpu.VMEM((nblk, K_BLK), jnp.float32),
            pltpu.VMEM((2, K_BLK, d), jnp.float32),
            pltpu.VMEM((2, K_BLK, d), jnp.float32),
            pltpu.VMEM((ZR, d), jnp.float32),
            pltpu.VMEM_SHARED((n, d), jnp.float32),
            pltpu.SemaphoreType.DMA,
            pltpu.SemaphoreType.DMA,
            pltpu.SemaphoreType.DMA,
            pltpu.SemaphoreType.DMA,
        ],
        compiler_params=pltpu.CompilerParams(use_tc_tiling_on_sc=False),
    )
    def edge_kernel(g_hbm, row_hbm, col_hbm, w_hbm, out_hbm,
                    row_v, col_v, w_v, rows_v, sc_v, zed_v, acc_sh,
                    gsem0, gsem1, ssem0, ssem1):
        gsem = (gsem0, gsem1)
        ssem = (ssem0, ssem1)
        c = lax.axis_index("c")
        s = lax.axis_index("s")
        wid = c * NS + s

        _zero_vmem_2d(zed_v, ZR, d)

        @pl.when(s < NS - 1)
        def _():
            for off, sz in _chunk_list(ch, ZR):
                pltpu.sync_copy(zed_v.at[pl.ds(0, sz)],
                                acc_sh.at[pl.ds(s * ch + off, sz)])

        @pl.when(s == NS - 1)
        def _():
            for off, sz in _chunk_list(last, ZR):
                pltpu.sync_copy(zed_v.at[pl.ds(0, sz)],
                                acc_sh.at[pl.ds((NS - 1) * ch + off, sz)])

        pltpu.sync_copy(row_hbm.at[wid], row_v)
        pltpu.sync_copy(col_hbm.at[wid], col_v)
        pltpu.sync_copy(w_hbm.at[wid], w_v)
        plsc.subcore_barrier()

        for b in range(2):
            pltpu.async_copy(g_hbm.at[row_v.at[b]], rows_v.at[b], gsem[b])

        def blk2(j0, carry):
            for b in range(2):
                j = j0 * 2 + b
                pltpu.make_async_copy(g_hbm.at[row_v.at[j]], rows_v.at[b],
                                      gsem[b]).wait()

                @pl.when(j0 > 0)
                def _():
                    jp = j - 2
                    pltpu.make_async_copy(sc_v.at[b],
                                          acc_sh.at[col_v.at[jp]],
                                          ssem[b]).wait()

                def scale(kb, carry2):
                    w16 = w_v[j, pl.ds(kb * LANES, LANES)]
                    for jj in range(LANES):
                        wk = _lane_bcast(w16, jj)
                        k = kb * LANES + jj
                        for dd in range(d // LANES):
                            sl = pl.ds(dd * LANES, LANES)
                            sc_v[b, k, sl] = rows_v[b, k, sl] * wk
                    return carry2

                lax.fori_loop(0, K_BLK // LANES, scale, 0)
                pltpu.async_copy(sc_v.at[b], acc_sh.at[col_v.at[j]],
                                 ssem[b], add=True)

                @pl.when(j + 2 < nblk)
                def _():
                    pltpu.async_copy(g_hbm.at[row_v.at[j + 2]], rows_v.at[b],
                                     gsem[b])
            return carry

        lax.fori_loop(0, nblk // 2, blk2, 0)
        for b in range(2):
            pltpu.make_async_copy(sc_v.at[b],
                                  acc_sh.at[col_v.at[nblk - 2 + b]],
                                  ssem[b]).wait()
        plsc.subcore_barrier()

        osl = pl.ds(c * d, d)

        @pl.when(s < NS - 1)
        def _():
            for off, sz in _chunk_list(ch, ZR):
                pltpu.sync_copy(acc_sh.at[pl.ds(s * ch + off, sz)],
                                zed_v.at[pl.ds(0, sz)])
                pltpu.sync_copy(zed_v.at[pl.ds(0, sz)],
                                out_hbm.at[pl.ds(s * ch + off, sz), osl])

        @pl.when(s == NS - 1)
        def _():
            for off, sz in _chunk_list(last, ZR):
                pltpu.sync_copy(acc_sh.at[pl.ds((NS - 1) * ch + off, sz)],
                                zed_v.at[pl.ds(0, sz)])
                pltpu.sync_copy(zed_v.at[pl.ds(0, sz)],
                                out_hbm.at[pl.ds((NS - 1) * ch + off, sz), osl])

    return edge_kernel


GI = 8


def _make_edge1_kernel(n, nblk2, d):
    assert d % LANES == 0 and nblk2 % (2 * GI) == 0 and nblk2 // GI >= 2
    ch, last = _tile_slices(n)
    ngrp = nblk2 // GI
    mesh = plsc.VectorSubcoreMesh(core_axis_name="c", subcore_axis_name="s")

    @functools.partial(
        pl.kernel,
        out_type=jax.ShapeDtypeStruct((n, NC * d), jnp.float32),
        mesh=mesh,
        scratch_types=[
            pltpu.VMEM((2, GI, K_BLK), jnp.int32),
            pltpu.VMEM((2, GI, K_BLK), jnp.int32),
            pltpu.VMEM((2, GI, K_BLK), jnp.float32),
            pltpu.VMEM((2, K_BLK, d), jnp.float32),
            pltpu.VMEM((2, K_BLK, d), jnp.float32),
            pltpu.VMEM((ZR, d), jnp.float32),
            pltpu.VMEM_SHARED((n, d), jnp.float32),
            pltpu.SemaphoreType.DMA,
            pltpu.SemaphoreType.DMA,
            pltpu.SemaphoreType.DMA,
            pltpu.SemaphoreType.DMA,
            pltpu.SemaphoreType.DMA,
            pltpu.SemaphoreType.DMA,
        ],
        compiler_params=pltpu.CompilerParams(use_tc_tiling_on_sc=False),
    )
    def edge1_kernel(g_hbm, row_hbm, col_hbm, w_hbm, out_hbm,
                     row_v, col_v, w_v, rows_v, sc_v, zed_v, acc_sh,
                     gsem0, gsem1, ssem0, ssem1, isem0, isem1):
        gsem = (gsem0, gsem1)
        ssem = (ssem0, ssem1)
        isem = (isem0, isem1)
        c = lax.axis_index("c")
        s = lax.axis_index("s")
        gsl = pl.ds(c * d, d)
        cn16 = jnp.full((LANES,), c * n, jnp.int32)

        _zero_vmem_2d(zed_v, ZR, d)

        @pl.when(s < NS - 1)
        def _():
            for off, sz in _chunk_list(ch, ZR):
                pltpu.sync_copy(zed_v.at[pl.ds(0, sz)],
                                acc_sh.at[pl.ds(s * ch + off, sz)])

        @pl.when(s == NS - 1)
        def _():
            for off, sz in _chunk_list(last, ZR):
                pltpu.sync_copy(zed_v.at[pl.ds(0, sz)],
                                acc_sh.at[pl.ds((NS - 1) * ch + off, sz)])

        def offset_rows(buf):
            def obody(q, carry):
                r = q // (K_BLK // LANES)
                o = (q % (K_BLK // LANES)) * LANES
                sl = pl.ds(o, LANES)
                row_v[buf, r, sl] = row_v[buf, r, sl] + cn16
                return carry
            lax.fori_loop(0, GI * K_BLK // LANES, obody, 0)

        def stage_group(g, buf, sem):
            pltpu.async_copy(row_hbm.at[s, pl.ds(g * GI, GI)], row_v.at[buf], sem)
            pltpu.async_copy(col_hbm.at[s, pl.ds(g * GI, GI)], col_v.at[buf], sem)
            pltpu.async_copy(w_hbm.at[s, pl.ds(g * GI, GI)], w_v.at[buf], sem)

        def wait_group(g, buf, sem):
            pltpu.make_async_copy(row_hbm.at[s, pl.ds(g * GI, GI)],
                                  row_v.at[buf], sem).wait()
            pltpu.make_async_copy(col_hbm.at[s, pl.ds(g * GI, GI)],
                                  col_v.at[buf], sem).wait()
            pltpu.make_async_copy(w_hbm.at[s, pl.ds(g * GI, GI)],
                                  w_v.at[buf], sem).wait()

        stage_group(0, 0, isem[0])
        wait_group(0, 0, isem[0])
        offset_rows(0)
        plsc.subcore_barrier()

        for b in range(2):
            pltpu.async_copy(g_hbm.at[row_v.at[0, b]], rows_v.at[b], gsem[b])

        def grouppair(g0, carry):
            for bgi in range(2):
                g = g0 * 2 + bgi
                for jb in range(GI):
                    b = jb % 2
                    j = g * GI + jb
                    pltpu.make_async_copy(g_hbm.at[row_v.at[bgi, jb]],
                                          rows_v.at[b], gsem[b]).wait()
                    if jb >= 2:
                        cprev = col_v.at[bgi, jb - 2]
                    else:
                        cprev = col_v.at[1 - bgi, GI - 2 + jb]

                    @pl.when(j >= 2)
                    def _(cprev=cprev, b=b):
                        pltpu.make_async_copy(sc_v.at[b], acc_sh.at[cprev],
                                              ssem[b]).wait()

                    def scale(kb, carry2, bgi=bgi, jb=jb, b=b):
                        w16 = w_v[bgi, jb, pl.ds(kb * LANES, LANES)]
                        for jj in range(LANES):
                            wk = _lane_bcast(w16, jj)
                            k = kb * LANES + jj
                            for dd in range(d // LANES):
                                sl = pl.ds(dd * LANES, LANES)
                                sc_v[b, k, sl] = rows_v[b, k, sl] * wk
                        return carry2

                    lax.fori_loop(0, K_BLK // LANES, scale, 0)
                    pltpu.async_copy(sc_v.at[b], acc_sh.at[col_v.at[bgi, jb]],
                                     ssem[b], add=True)

                    if jb == 2:
                        @pl.when(g + 1 < ngrp)
                        def _(g=g, bgi=bgi):
                            stage_group(g + 1, 1 - bgi, isem[1 - bgi])

                    if jb == GI - 2:
                        @pl.when(g + 1 < ngrp)
                        def _(g=g, bgi=bgi):
                            wait_group(g + 1, 1 - bgi, isem[1 - bgi])
                            offset_rows(1 - bgi)

                    if jb <= GI - 3:
                        nidx = row_v.at[bgi, jb + 2]
                    else:
                        nidx = row_v.at[1 - bgi, jb + 2 - GI]

                    @pl.when(j + 2 < nblk2)
                    def _(nidx=nidx, b=b):
                        pltpu.async_copy(g_hbm.at[nidx], rows_v.at[b], gsem[b])
            return carry

        lax.fori_loop(0, ngrp // 2, grouppair, 0)
        for b in range(2):
            pltpu.make_async_copy(sc_v.at[b],
                                  acc_sh.at[col_v.at[1, GI - 2 + b]],
                                  ssem[b]).wait()
        plsc.subcore_barrier()

        @pl.when(s < NS - 1)
        def _():
            for off, sz in _chunk_list(ch, ZR):
                pltpu.sync_copy(acc_sh.at[pl.ds(s * ch + off, sz)],
                                zed_v.at[pl.ds(0, sz)])
                pltpu.sync_copy(zed_v.at[pl.ds(0, sz)],
                                out_hbm.at[pl.ds(s * ch + off, sz), gsl])

        @pl.when(s == NS - 1)
        def _():
            for off, sz in _chunk_list(last, ZR):
                pltpu.sync_copy(acc_sh.at[pl.ds((NS - 1) * ch + off, sz)],
                                zed_v.at[pl.ds(0, sz)])
                pltpu.sync_copy(zed_v.at[pl.ds(0, sz)],
                                out_hbm.at[pl.ds((NS - 1) * ch + off, sz), gsl])

    return edge1_kernel


def _dis_from(deg_ref):
    deg = deg_ref[:, 0] + deg_ref[:, 1] + 1.0
    return jnp.where(deg > 0, lax.rsqrt(deg), 0.0)


def _make_prep_body(dch):
    def body(deg_ref, x_ref, w_ref, gf_ref, gcm_ref):
        dis = _dis_from(deg_ref)
        h = jnp.dot(x_ref[...], w_ref[...], preferred_element_type=jnp.float32)
        g = h * dis[:, None]
        gf_ref[...] = g
        for p in range(NC):
            gcm_ref[p] = g[:, p * dch:(p + 1) * dch]
    return body


def _mid_body(deg_ref, s_ref, g1_ref, b1_ref, w2_ref, g2_ref):
    dis = _dis_from(deg_ref)
    t = (s_ref[...] + g1_ref[...]) * dis[:, None] + b1_ref[...]
    t = jnp.maximum(t, 0.0)
    h2 = jnp.dot(t, w2_ref[...], preferred_element_type=jnp.float32)
    g2_ref[...] = h2 * dis[:, None]


def _make_fin_body(d2):
    def body(deg_ref, s_ref, g2_ref, b2_ref, o_ref):
        dis = _dis_from(deg_ref)
        s = s_ref[:, :d2] + s_ref[:, d2:NC * d2]
        o_ref[...] = (s + g2_ref[...]) * dis[:, None] + b2_ref[...]
    return body


def _row_blocks(n):
    for blk in (2000, 1000, 500, 250, 125, n):
        if n % blk == 0 and blk % 8 == 0:
            return blk, n // blk
    return n, 1


def kernel(x, edge_index, edge_weight, W1, b1, W2, b2):
    n, f = x.shape
    d1 = W1.shape[1]
    d2 = W2.shape[1]
    e = edge_weight.shape[0]

    row = edge_index[0].astype(jnp.int32)
    col = edge_index[1].astype(jnp.int32)
    w = edge_weight.astype(jnp.float32)

    emult = NS * K_BLK * GI * 2
    e_pad = (e + emult - 1) // emult * emult
    if e_pad != e:
        extra = e_pad - e
        pad_idx = (jnp.arange(extra, dtype=jnp.int32) * 16) % n
        row = jnp.concatenate([row, pad_idx])
        col = jnp.concatenate([col, pad_idx])
        w = jnp.concatenate([w, jnp.zeros((extra,), jnp.float32)])
    nblk = e_pad // (NW * K_BLK)
    nblk2 = e_pad // (NS * K_BLK)
    row3 = row.reshape(NW, nblk, K_BLK)
    col3 = col.reshape(NW, nblk, K_BLK)
    w3 = w.reshape(NW, nblk, K_BLK)
    row2 = row.reshape(NS, nblk2, K_BLK)
    col2 = col.reshape(NS, nblk2, K_BLK)
    w2 = w.reshape(NS, nblk2, K_BLK)

    deg_parts = _make_deg_kernel(n, nblk)(col3, w3)
    deg_nt = deg_parts.reshape(NC, n).T

    blk, nrblk = _row_blocks(n)
    full2 = lambda i: (0, 0)

    DCH = d1 // NC
    assert d1 == NC * DCH and d2 == DCH

    g1f, g1cm = pl.pallas_call(
        _make_prep_body(DCH),
        grid=(nrblk,),
        in_specs=[
            pl.BlockSpec((blk, NC), lambda i: (i, 0)),
            pl.BlockSpec((blk, f), lambda i: (i, 0)),
            pl.BlockSpec((f, d1), full2),
        ],
        out_specs=[
            pl.BlockSpec((blk, d1), lambda i: (i, 0)),
            pl.BlockSpec((NC, blk, DCH), lambda i: (0, i, 0)),
        ],
        out_shape=[
            jax.ShapeDtypeStruct((n, d1), jnp.float32),
            jax.ShapeDtypeStruct((NC, n, DCH), jnp.float32),
        ],
    )(deg_nt, x, W1)

    s1 = _make_edge1_kernel(n, nblk2, DCH)(
        g1cm.reshape(NC * n, DCH), row2, col2, w2)

    g2 = pl.pallas_call(
        _mid_body,
        grid=(nrblk,),
        in_specs=[
            pl.BlockSpec((blk, NC), lambda i: (i, 0)),
            pl.BlockSpec((blk, d1), lambda i: (i, 0)),
            pl.BlockSpec((blk, d1), lambda i: (i, 0)),
            pl.BlockSpec((1, d1), full2),
            pl.BlockSpec((d1, d2), full2),
        ],
        out_specs=pl.BlockSpec((blk, d2), lambda i: (i, 0)),
        out_shape=jax.ShapeDtypeStruct((n, d2), jnp.float32),
    )(deg_nt, s1, g1f, b1.reshape(1, d1), W2)

    s2 = _make_edge_kernel(n, nblk, d2)(g2, row3, col3, w3)

    out = pl.pallas_call(
        _make_fin_body(d2),
        grid=(nrblk,),
        in_specs=[
            pl.BlockSpec((blk, NC), lambda i: (i, 0)),
            pl.BlockSpec((blk, NC * d2), lambda i: (i, 0)),
            pl.BlockSpec((blk, d2), lambda i: (i, 0)),
            pl.BlockSpec((1, d2), full2),
        ],
        out_specs=pl.BlockSpec((blk, d2), lambda i: (i, 0)),
        out_shape=jax.ShapeDtypeStruct((n, d2), jnp.float32),
    )(deg_nt, s2, g2, b2.reshape(1, d2))

    return out
